# baseline (speedup 1.0000x reference)
import functools

import jax
import jax.numpy as jnp
from jax import lax
from jax.experimental import pallas as pl
from jax.experimental.pallas import tpu as pltpu
from jax.experimental.pallas import tpu_sc as plsc

_N_EMBED = 1024
_EMBED_DIM = 64
_BETA = 0.25
_ROWS = 16384
_TILE = 1024
_GRID = _ROWS // _TILE

_NC = 2
_NS = 16
_NW = _NC * _NS
_RPW = _ROWS // _NW
_CHUNK = 128


def _l2n(x, eps=1e-12):
    n = jnp.linalg.norm(x, ord=2, axis=-1, keepdims=True)
    return x / jnp.maximum(n, eps)


_HGRID = _ROWS // 2 // _TILE


def _tc_body(z_ref, ent_ref, esq_ref, idx_ref, loss_ref):
    i = pl.program_id(0)
    z = z_ref[...]
    nrm = jnp.sqrt(jnp.sum(z * z, axis=1, keepdims=True))
    zn = z / jnp.maximum(nrm, 1e-12)
    zsq = jnp.sum(zn * zn, axis=1, keepdims=True)
    s = lax.dot_general(zn, ent_ref[...], (((1,), (0,)), ((), ())),
                        precision=lax.Precision.DEFAULT,
                        preferred_element_type=jnp.float32)
    d2 = (zsq - 2.0 * s) + esq_ref[...]
    dist = jnp.sqrt(jnp.maximum(d2, 0.0))
    m = jnp.min(dist, axis=1, keepdims=True)
    iota = lax.broadcasted_iota(jnp.int32, (_TILE, _N_EMBED), 1)
    big = jnp.int32(_N_EMBED)
    idx = jnp.min(jnp.where(dist == m, iota, big), axis=1, keepdims=True)
    idx_ref[...] = idx

    part = jnp.sum(m * m)

    @pl.when(i == 0)
    def _init():
        loss_ref[0, 0] = 0.0

    loss_ref[0, 0] += part

    @pl.when(i == _HGRID - 1)
    def _fin():
        loss_ref[0, 0] = loss_ref[0, 0] * jnp.float32(
            _BETA / (_ROWS * _EMBED_DIM))


_HROWS = _ROWS // 2
_HRPW = _HROWS // _NW


def _sc_body(emb_hbm, idx_hbm, out_hbm, idx_v, q_v, sem):
    wid = lax.axis_index("s") * _NC + lax.axis_index("c")
    base = wid * _HRPW
    pltpu.sync_copy(idx_hbm.at[pl.ds(base, _HRPW)], idx_v)
    cps = []
    for j in range(_HRPW // _CHUNK):
        cps.append(pltpu.async_copy(
            emb_hbm.at[idx_v.at[pl.ds(j * _CHUNK, _CHUNK)]],
            q_v.at[pl.ds(j * _CHUNK, _CHUNK)], sem))
    for cp in cps:
        cp.wait()
    pltpu.sync_copy(q_v, out_hbm.at[pl.ds(base, _HRPW)])


@functools.cache
def _sc_gather():
    return pl.kernel(
        _sc_body,
        mesh=plsc.VectorSubcoreMesh(core_axis_name="c",
                                    subcore_axis_name="s"),
        out_type=jax.ShapeDtypeStruct((_HROWS, 2 * _EMBED_DIM),
                                      jnp.float32),
        scratch_types=[
            pltpu.VMEM((_HRPW,), jnp.int32),
            pltpu.VMEM((_HRPW, 2 * _EMBED_DIM), jnp.float32),
            pltpu.SemaphoreType.DMA,
        ],
    )


def _st_body(z_ref, qa_ref, qb_ref, out_ref):
    h = pl.program_id(0)
    z = z_ref[...]
    qp = jnp.where(h == 0, qa_ref[...], qb_ref[...])
    q = qp[:, :_EMBED_DIM]
    out_ref[...] = z + (q - z)


@jax.jit
def kernel(z, embed_weight):
    flat_z = z.reshape(-1, _EMBED_DIM)
    en = _l2n(embed_weight)
    esq = jnp.sum(en * en, axis=1)[None, :]
    ent = en.T

    grid_spec = pl.GridSpec(
        grid=(_HGRID,),
        in_specs=[
            pl.BlockSpec((_TILE, _EMBED_DIM), lambda i: (i, 0)),
            pl.BlockSpec((_EMBED_DIM, _N_EMBED), lambda i: (0, 0)),
            pl.BlockSpec((1, _N_EMBED), lambda i: (0, 0)),
        ],
        out_specs=[
            pl.BlockSpec((_TILE, 1), lambda i: (i, 0)),
            pl.BlockSpec(memory_space=pltpu.SMEM),
        ],
    )
    dist_call = pl.pallas_call(
        _tc_body,
        grid_spec=grid_spec,
        out_shape=[
            jax.ShapeDtypeStruct((_HROWS, 1), jnp.int32),
            jax.ShapeDtypeStruct((1, 1), jnp.float32),
        ],
    )

    idx_a, loss_a = dist_call(flat_z[:_HROWS], ent, esq)
    idx_b, loss_b = dist_call(flat_z[_HROWS:], ent, esq)

    embp = jnp.pad(embed_weight, ((0, 0), (0, _EMBED_DIM)))
    qa = _sc_gather()(embp, idx_a.reshape(_HROWS))
    qb = _sc_gather()(embp, idx_b.reshape(_HROWS))
    idx = jnp.concatenate([idx_a, idx_b], axis=0)
    loss = loss_a + loss_b

    hgrid = _HROWS // _TILE
    qst = pl.pallas_call(
        _st_body,
        grid=(2, hgrid),
        in_specs=[
            pl.BlockSpec((_TILE, _EMBED_DIM), lambda h, i: (h * hgrid + i, 0)),
            pl.BlockSpec((_TILE, 2 * _EMBED_DIM), lambda h, i: (i, 0)),
            pl.BlockSpec((_TILE, 2 * _EMBED_DIM), lambda h, i: (i, 0)),
        ],
        out_specs=pl.BlockSpec((_TILE, _EMBED_DIM),
                               lambda h, i: (h * hgrid + i, 0)),
        out_shape=jax.ShapeDtypeStruct((_ROWS, _EMBED_DIM), jnp.float32),
    )(flat_z, qa, qb)

    quantized_st = qst.reshape(z.shape)
    encoding_indices = idx.reshape(z.shape[:-1])
    vq_loss = loss[0, 0]
    return quantized_st, encoding_indices, vq_loss


# speedup vs baseline: 1.0731x; 1.0731x over previous
import functools

import jax
import jax.numpy as jnp
from jax import lax
from jax.experimental import pallas as pl
from jax.experimental.pallas import tpu as pltpu
from jax.experimental.pallas import tpu_sc as plsc

_N_EMBED = 1024
_EMBED_DIM = 64
_BETA = 0.25
_ROWS = 16384
_TILE = 1024
_GRID = _ROWS // _TILE

_NC = 2
_NS = 16
_NW = _NC * _NS
_RPW = _ROWS // _NW
_CHUNK = 128


def _l2n(x, eps=1e-12):
    n = jnp.linalg.norm(x, ord=2, axis=-1, keepdims=True)
    return x / jnp.maximum(n, eps)


_HGRID = _ROWS // 2 // _TILE


def _tc_body(z_ref, ent_ref, esq_ref, idx_ref, loss_ref):
    i = pl.program_id(0)
    z = z_ref[...]
    nrm = jnp.sqrt(jnp.sum(z * z, axis=1, keepdims=True))
    zn = z / jnp.maximum(nrm, 1e-12)
    zsq = jnp.sum(zn * zn, axis=1, keepdims=True)
    s = lax.dot_general(zn, ent_ref[...], (((1,), (0,)), ((), ())),
                        precision=lax.Precision.DEFAULT,
                        preferred_element_type=jnp.float32)
    d2 = (zsq - 2.0 * s) + esq_ref[...]
    dist = jnp.sqrt(jnp.maximum(d2, 0.0))
    m = jnp.min(dist, axis=1, keepdims=True)
    iota = lax.broadcasted_iota(jnp.int32, (_TILE, _N_EMBED), 1)
    big = jnp.int32(_N_EMBED)
    idx = jnp.min(jnp.where(dist == m, iota, big), axis=1, keepdims=True)
    idx_ref[...] = idx

    part = jnp.sum(m * m)

    @pl.when(i == 0)
    def _init():
        loss_ref[0, 0] = 0.0

    loss_ref[0, 0] += part

    @pl.when(i == _GRID - 1)
    def _fin():
        loss_ref[0, 0] = loss_ref[0, 0] * jnp.float32(
            _BETA / (_ROWS * _EMBED_DIM))


_HROWS = _ROWS // 2
_HRPW = _HROWS // _NW


def _sc_body(emb_hbm, idx_hbm, out_hbm, idx_v, q_v, sem):
    wid = lax.axis_index("s") * _NC + lax.axis_index("c")
    base = wid * _HRPW
    pltpu.sync_copy(idx_hbm.at[pl.ds(base, _HRPW)], idx_v)
    cps = []
    for j in range(_HRPW // _CHUNK):
        cps.append(pltpu.async_copy(
            emb_hbm.at[idx_v.at[pl.ds(j * _CHUNK, _CHUNK)]],
            q_v.at[pl.ds(j * _CHUNK, _CHUNK)], sem))
    for cp in cps:
        cp.wait()
    pltpu.sync_copy(q_v, out_hbm.at[pl.ds(base, _HRPW)])


@functools.cache
def _sc_gather():
    return pl.kernel(
        _sc_body,
        mesh=plsc.VectorSubcoreMesh(core_axis_name="c",
                                    subcore_axis_name="s"),
        out_type=jax.ShapeDtypeStruct((_HROWS, 2 * _EMBED_DIM),
                                      jnp.float32),
        scratch_types=[
            pltpu.VMEM((_HRPW,), jnp.int32),
            pltpu.VMEM((_HRPW, 2 * _EMBED_DIM), jnp.float32),
            pltpu.SemaphoreType.DMA,
        ],
    )


def _st_body(z_ref, qa_ref, qb_ref, out_ref):
    h = pl.program_id(0)
    z = z_ref[...]
    qp = jnp.where(h == 0, qa_ref[...], qb_ref[...])
    q = qp[:, :_EMBED_DIM]
    out_ref[...] = z + (q - z)


@jax.jit
def kernel(z, embed_weight):
    flat_z = z.reshape(-1, _EMBED_DIM)
    en = _l2n(embed_weight)
    esq = jnp.sum(en * en, axis=1)[None, :]
    ent = en.T

    grid_spec = pl.GridSpec(
        grid=(_GRID,),
        in_specs=[
            pl.BlockSpec((_TILE, _EMBED_DIM), lambda i: (i, 0)),
            pl.BlockSpec((_EMBED_DIM, _N_EMBED), lambda i: (0, 0)),
            pl.BlockSpec((1, _N_EMBED), lambda i: (0, 0)),
        ],
        out_specs=[
            pl.BlockSpec((_TILE, 1), lambda i: (i, 0)),
            pl.BlockSpec(memory_space=pltpu.SMEM),
        ],
    )
    idx, loss = pl.pallas_call(
        _tc_body,
        grid_spec=grid_spec,
        out_shape=[
            jax.ShapeDtypeStruct((_ROWS, 1), jnp.int32),
            jax.ShapeDtypeStruct((1, 1), jnp.float32),
        ],
    )(flat_z, ent, esq)

    embp = jnp.pad(embed_weight, ((0, 0), (0, _EMBED_DIM)))
    idx1d = idx.reshape(_ROWS)
    qa = _sc_gather()(embp, idx1d[:_HROWS])
    qb = _sc_gather()(embp, idx1d[_HROWS:])

    hgrid = _HROWS // _TILE
    qst = pl.pallas_call(
        _st_body,
        grid=(2, hgrid),
        in_specs=[
            pl.BlockSpec((_TILE, _EMBED_DIM), lambda h, i: (h * hgrid + i, 0)),
            pl.BlockSpec((_TILE, 2 * _EMBED_DIM), lambda h, i: (i, 0)),
            pl.BlockSpec((_TILE, 2 * _EMBED_DIM), lambda h, i: (i, 0)),
        ],
        out_specs=pl.BlockSpec((_TILE, _EMBED_DIM),
                               lambda h, i: (h * hgrid + i, 0)),
        out_shape=jax.ShapeDtypeStruct((_ROWS, _EMBED_DIM), jnp.float32),
    )(flat_z, qa, qb)

    quantized_st = qst.reshape(z.shape)
    encoding_indices = idx.reshape(z.shape[:-1])
    vq_loss = loss[0, 0]
    return quantized_st, encoding_indices, vq_loss
